# baseline (speedup 1.0000x reference)
import jax
import jax.numpy as jnp
import numpy as np
from jax.experimental import pallas as pl
from jax.experimental.pallas import tpu as pltpu
from jax.sharding import Mesh, PartitionSpec as P

_BN = 1024
_BF = 1024


def _moe_body(x_ref, wg_ref, bg_ref, we_ref, be_ref, out_ref, g_scr):
    f = pl.program_id(1)
    e = pl.program_id(2)
    n_exp = g_scr.shape[1]

    @pl.when((f == 0) & (e == 0))
    def _prep():
        logits = jnp.dot(x_ref[...], wg_ref[...],
                         preferred_element_type=jnp.float32)
        logits = logits + bg_ref[...]
        m = jnp.max(logits, axis=-1, keepdims=True)
        p = jnp.exp(logits - m)
        g_scr[...] = p / jnp.sum(p, axis=-1, keepdims=True)

    lane = jax.lax.broadcasted_iota(jnp.int32, (1, n_exp), 1)
    ge = jnp.sum(jnp.where(lane == e, g_scr[...], 0.0), axis=-1, keepdims=True)

    mm = jnp.dot(x_ref[...], we_ref[0],
                 preferred_element_type=jnp.float32)
    contrib = ge * (mm + be_ref[0])

    @pl.when(e == 0)
    def _init():
        out_ref[...] = contrib

    @pl.when(e != 0)
    def _acc():
        out_ref[...] += contrib


def _moe_local(x, Wg, bg, We, be):
    n, k = x.shape
    n_exp = Wg.shape[1]
    f_out = We.shape[2]
    bn = min(_BN, n)
    bf = min(_BF, f_out)
    grid = (n // bn, f_out // bf, n_exp)
    return pl.pallas_call(
        _moe_body,
        grid=grid,
        in_specs=[
            pl.BlockSpec((bn, k), lambda i, f, e: (i, 0)),
            pl.BlockSpec((k, n_exp), lambda i, f, e: (0, 0)),
            pl.BlockSpec((1, n_exp), lambda i, f, e: (0, 0)),
            pl.BlockSpec((1, k, bf), lambda i, f, e: (e, 0, f)),
            pl.BlockSpec((1, 1, bf), lambda i, f, e: (e, 0, f)),
        ],
        out_specs=pl.BlockSpec((bn, bf), lambda i, f, e: (i, f)),
        out_shape=jax.ShapeDtypeStruct((n, f_out), jnp.float32),
        scratch_shapes=[
            pltpu.VMEM((bn, n_exp), jnp.float32),
        ],
        compiler_params=pltpu.CompilerParams(
            dimension_semantics=("parallel", "parallel", "arbitrary"),
        ),
    )(x, Wg, bg.reshape(1, n_exp), We, be.reshape(n_exp, 1, f_out))


def kernel(x, Wg, bg, We, be):
    f_out = We.shape[2]
    devs = jax.devices()
    ndev = 2 if len(devs) >= 2 and f_out % (2 * 256) == 0 else 1
    mesh = Mesh(np.array(devs[:ndev]), ("fx",))
    fn = jax.shard_map(
        _moe_local,
        mesh=mesh,
        in_specs=(P(), P(), P(), P(None, None, "fx"), P(None, "fx")),
        out_specs=P(None, "fx"),
        check_vma=False,
    )
    return fn(x.astype(jnp.bfloat16), Wg.astype(jnp.bfloat16), bg,
              We.astype(jnp.bfloat16), be)


# speedup vs baseline: 1.8773x; 1.8773x over previous
import jax
import jax.numpy as jnp
import numpy as np
from jax.experimental import pallas as pl
from jax.experimental.pallas import tpu as pltpu
from jax.sharding import Mesh, PartitionSpec as P

_BN = 1024
_BF = 1024


def _moe_body(x_ref, wg_ref, bg_ref, we_ref, be_ref, out_ref, g_scr):
    f = pl.program_id(1)
    e = pl.program_id(2)
    n_exp = g_scr.shape[1]

    @pl.when((f == 0) & (e == 0))
    def _prep():
        logits = jnp.dot(x_ref[...], wg_ref[...],
                         preferred_element_type=jnp.float32)
        logits = logits + bg_ref[...]
        m = jnp.max(logits, axis=-1, keepdims=True)
        p = jnp.exp(logits - m)
        g_scr[...] = p / jnp.sum(p, axis=-1, keepdims=True)

    lane = jax.lax.broadcasted_iota(jnp.int32, (1, n_exp), 1)
    ge = jnp.sum(jnp.where(lane == e, g_scr[...], 0.0), axis=-1, keepdims=True)

    mm = jnp.dot(x_ref[...], we_ref[0],
                 preferred_element_type=jnp.float32)
    contrib = ge * (mm + be_ref[0])

    @pl.when(e == 0)
    def _init():
        out_ref[...] = contrib

    @pl.when(e != 0)
    def _acc():
        out_ref[...] += contrib


def _moe_local(x, Wg, bg, We, be):
    n, k = x.shape
    n_exp = Wg.shape[1]
    f_out = We.shape[2]
    bn = min(_BN, n)
    bf = min(_BF, f_out)
    grid = (n // bn, f_out // bf, n_exp)
    return pl.pallas_call(
        _moe_body,
        grid=grid,
        in_specs=[
            pl.BlockSpec((bn, k), lambda i, f, e: (i, 0)),
            pl.BlockSpec((k, n_exp), lambda i, f, e: (0, 0)),
            pl.BlockSpec((1, n_exp), lambda i, f, e: (0, 0)),
            pl.BlockSpec((1, k, bf), lambda i, f, e: (e, 0, f)),
            pl.BlockSpec((1, 1, bf), lambda i, f, e: (e, 0, f)),
        ],
        out_specs=pl.BlockSpec((bn, bf), lambda i, f, e: (i, f)),
        out_shape=jax.ShapeDtypeStruct((n, f_out), jnp.float32),
        scratch_shapes=[
            pltpu.VMEM((bn, n_exp), jnp.float32),
        ],
        compiler_params=pltpu.CompilerParams(
            dimension_semantics=("parallel", "parallel", "arbitrary"),
        ),
    )(x, Wg, bg.reshape(1, n_exp), We, be.reshape(n_exp, 1, f_out))


def kernel(x, Wg, bg, We, be):
    return _moe_local(x.astype(jnp.bfloat16), Wg.astype(jnp.bfloat16), bg,
                      We.astype(jnp.bfloat16), be)


# speedup vs baseline: 2.2640x; 1.2060x over previous
import jax
import jax.numpy as jnp
from jax.experimental import pallas as pl
from jax.experimental.pallas import tpu as pltpu

_BN = 1024
_BF = 1024


def _moe_body(x_ref, wg_ref, bg_ref, we_ref, be_ref, out_ref, g_scr, xb_scr):
    f = pl.program_id(1)
    e = pl.program_id(2)
    n_exp = g_scr.shape[1]

    @pl.when((f == 0) & (e == 0))
    def _prep():
        xf = x_ref[...]
        logits = jnp.dot(xf, wg_ref[...], preferred_element_type=jnp.float32)
        logits = logits + bg_ref[...]
        m = jnp.max(logits, axis=-1, keepdims=True)
        p = jnp.exp(logits - m)
        g_scr[...] = p / jnp.sum(p, axis=-1, keepdims=True)
        xb_scr[...] = xf.astype(jnp.bfloat16)

    lane = jax.lax.broadcasted_iota(jnp.int32, (1, n_exp), 1)
    ge = jnp.sum(jnp.where(lane == e, g_scr[...], 0.0), axis=-1, keepdims=True)

    mm = jnp.dot(xb_scr[...], we_ref[0].astype(jnp.bfloat16),
                 preferred_element_type=jnp.float32)
    contrib = ge * (mm + be_ref[0])

    @pl.when(e == 0)
    def _init():
        out_ref[...] = contrib

    @pl.when(e != 0)
    def _acc():
        out_ref[...] += contrib


def kernel(x, Wg, bg, We, be):
    n, k = x.shape
    n_exp = Wg.shape[1]
    f_out = We.shape[2]
    bn = min(_BN, n)
    bf = min(_BF, f_out)
    grid = (n // bn, f_out // bf, n_exp)
    return pl.pallas_call(
        _moe_body,
        grid=grid,
        in_specs=[
            pl.BlockSpec((bn, k), lambda i, f, e: (i, 0)),
            pl.BlockSpec((k, n_exp), lambda i, f, e: (0, 0)),
            pl.BlockSpec((1, n_exp), lambda i, f, e: (0, 0)),
            pl.BlockSpec((1, k, bf), lambda i, f, e: (e, 0, f)),
            pl.BlockSpec((1, 1, bf), lambda i, f, e: (e, 0, f)),
        ],
        out_specs=pl.BlockSpec((bn, bf), lambda i, f, e: (i, f)),
        out_shape=jax.ShapeDtypeStruct((n, f_out), jnp.float32),
        scratch_shapes=[
            pltpu.VMEM((bn, n_exp), jnp.float32),
            pltpu.VMEM((bn, k), jnp.bfloat16),
        ],
        compiler_params=pltpu.CompilerParams(
            dimension_semantics=("parallel", "parallel", "arbitrary"),
        ),
    )(x, Wg, bg.reshape(1, n_exp), We, be.reshape(n_exp, 1, f_out))


# speedup vs baseline: 2.2729x; 1.0040x over previous
import jax
import jax.numpy as jnp
from jax.experimental import pallas as pl
from jax.experimental.pallas import tpu as pltpu

_BN = 1024
_BF = 1024


def _moe_body(x_ref, wg_ref, bg_ref, we_ref, be_ref, out_ref, g_scr):
    f = pl.program_id(1)
    e2 = pl.program_id(2)
    n_exp = g_scr.shape[1]

    @pl.when((f == 0) & (e2 == 0))
    def _prep():
        logits = jnp.dot(x_ref[...], wg_ref[...],
                         preferred_element_type=jnp.float32)
        logits = logits + bg_ref[...]
        m = jnp.max(logits, axis=-1, keepdims=True)
        p = jnp.exp(logits - m)
        g_scr[...] = p / jnp.sum(p, axis=-1, keepdims=True)

    lane = jax.lax.broadcasted_iota(jnp.int32, (1, n_exp), 1)
    g = g_scr[...]
    ge0 = jnp.sum(jnp.where(lane == 2 * e2, g, 0.0), axis=-1, keepdims=True)
    ge1 = jnp.sum(jnp.where(lane == 2 * e2 + 1, g, 0.0), axis=-1, keepdims=True)

    xb = x_ref[...]
    mm0 = jnp.dot(xb, we_ref[0].astype(jnp.bfloat16),
                  preferred_element_type=jnp.float32)
    mm1 = jnp.dot(xb, we_ref[1].astype(jnp.bfloat16),
                  preferred_element_type=jnp.float32)
    contrib = ge0 * (mm0 + be_ref[0]) + ge1 * (mm1 + be_ref[1])

    @pl.when(e2 == 0)
    def _init():
        out_ref[...] = contrib

    @pl.when(e2 != 0)
    def _acc():
        out_ref[...] += contrib


def kernel(x, Wg, bg, We, be):
    n, k = x.shape
    n_exp = Wg.shape[1]
    f_out = We.shape[2]
    bn = min(_BN, n)
    bf = min(_BF, f_out)
    grid = (n // bn, f_out // bf, n_exp // 2)
    return pl.pallas_call(
        _moe_body,
        grid=grid,
        in_specs=[
            pl.BlockSpec((bn, k), lambda i, f, e: (i, 0)),
            pl.BlockSpec((k, n_exp), lambda i, f, e: (0, 0)),
            pl.BlockSpec((1, n_exp), lambda i, f, e: (0, 0)),
            pl.BlockSpec((2, k, bf), lambda i, f, e: (e, 0, f)),
            pl.BlockSpec((2, 1, bf), lambda i, f, e: (e, 0, f)),
        ],
        out_specs=pl.BlockSpec((bn, bf), lambda i, f, e: (i, f)),
        out_shape=jax.ShapeDtypeStruct((n, f_out), jnp.float32),
        scratch_shapes=[
            pltpu.VMEM((bn, n_exp), jnp.float32),
        ],
        compiler_params=pltpu.CompilerParams(
            dimension_semantics=("parallel", "parallel", "arbitrary"),
        ),
    )(x.astype(jnp.bfloat16), Wg.astype(jnp.bfloat16), bg.reshape(1, n_exp),
      We, be.reshape(n_exp, 1, f_out))


# speedup vs baseline: 2.2793x; 1.0028x over previous
import jax
import jax.numpy as jnp
from jax.experimental import pallas as pl
from jax.experimental.pallas import tpu as pltpu

_BN = 1024
_BF = 1024


def _moe_body(x_ref, wg_ref, bg_ref, we_ref, be_ref, out_ref, g_scr):
    f = pl.program_id(1)
    e2 = pl.program_id(2)
    n_exp = g_scr.shape[1]

    @pl.when((f == 0) & (e2 == 0))
    def _prep():
        logits = jnp.dot(x_ref[...], wg_ref[...],
                         preferred_element_type=jnp.float32)
        logits = logits + bg_ref[...]
        m = jnp.max(logits, axis=-1, keepdims=True)
        p = jnp.exp(logits - m)
        g_scr[...] = p / jnp.sum(p, axis=-1, keepdims=True)

    lane = jax.lax.broadcasted_iota(jnp.int32, (1, n_exp), 1)
    g = g_scr[...]
    ge0 = jnp.sum(jnp.where(lane == 2 * e2, g, 0.0), axis=-1, keepdims=True)
    ge1 = jnp.sum(jnp.where(lane == 2 * e2 + 1, g, 0.0), axis=-1, keepdims=True)

    xb = x_ref[...]
    mm0 = jnp.dot(xb, we_ref[0].astype(jnp.bfloat16),
                  preferred_element_type=jnp.float32)
    mm1 = jnp.dot(xb, we_ref[1].astype(jnp.bfloat16),
                  preferred_element_type=jnp.float32)
    contrib = ge0 * mm0 + ge1 * mm1

    @pl.when(e2 == 0)
    def _init():
        bias = jnp.dot(g, be_ref[...], preferred_element_type=jnp.float32)
        out_ref[...] = contrib + bias

    @pl.when(e2 != 0)
    def _acc():
        out_ref[...] += contrib


def kernel(x, Wg, bg, We, be):
    n, k = x.shape
    n_exp = Wg.shape[1]
    f_out = We.shape[2]
    bn = min(_BN, n)
    bf = min(_BF, f_out)
    grid = (n // bn, f_out // bf, n_exp // 2)
    return pl.pallas_call(
        _moe_body,
        grid=grid,
        in_specs=[
            pl.BlockSpec((bn, k), lambda i, f, e: (i, 0)),
            pl.BlockSpec((k, n_exp), lambda i, f, e: (0, 0)),
            pl.BlockSpec((1, n_exp), lambda i, f, e: (0, 0)),
            pl.BlockSpec((2, k, bf), lambda i, f, e: (e, 0, f)),
            pl.BlockSpec((n_exp, bf), lambda i, f, e: (0, f)),
        ],
        out_specs=pl.BlockSpec((bn, bf), lambda i, f, e: (i, f)),
        out_shape=jax.ShapeDtypeStruct((n, f_out), jnp.float32),
        scratch_shapes=[
            pltpu.VMEM((bn, n_exp), jnp.float32),
        ],
        compiler_params=pltpu.CompilerParams(
            dimension_semantics=("parallel", "parallel", "arbitrary"),
        ),
    )(x.astype(jnp.bfloat16), Wg.astype(jnp.bfloat16), bg.reshape(1, n_exp),
      We, be)


# speedup vs baseline: 2.3194x; 1.0176x over previous
import jax
import jax.numpy as jnp
from jax.experimental import pallas as pl
from jax.experimental.pallas import tpu as pltpu

_BN = 1024
_BF = 512


def _moe_body(x_ref, wg_ref, bg_ref, we_ref, be_ref, out_ref, g_scr):
    f = pl.program_id(1)
    e2 = pl.program_id(2)
    n_exp = g_scr.shape[1]

    @pl.when((f == 0) & (e2 == 0))
    def _prep():
        logits = jnp.dot(x_ref[...], wg_ref[...],
                         preferred_element_type=jnp.float32)
        logits = logits + bg_ref[...]
        m = jnp.max(logits, axis=-1, keepdims=True)
        p = jnp.exp(logits - m)
        g_scr[...] = p / jnp.sum(p, axis=-1, keepdims=True)

    lane = jax.lax.broadcasted_iota(jnp.int32, (1, n_exp), 1)
    g = g_scr[...]
    xb = x_ref[...]
    contrib = 0.0
    for j in range(4):
        gej = jnp.sum(jnp.where(lane == 4 * e2 + j, g, 0.0),
                      axis=-1, keepdims=True)
        mmj = jnp.dot(xb, we_ref[j].astype(jnp.bfloat16),
                      preferred_element_type=jnp.float32)
        contrib = contrib + gej * mmj

    @pl.when(e2 == 0)
    def _init():
        bias = jnp.dot(g, be_ref[...], preferred_element_type=jnp.float32)
        out_ref[...] = contrib + bias

    @pl.when(e2 != 0)
    def _acc():
        out_ref[...] += contrib


def kernel(x, Wg, bg, We, be):
    n, k = x.shape
    n_exp = Wg.shape[1]
    f_out = We.shape[2]
    bn = min(_BN, n)
    bf = min(_BF, f_out)
    grid = (n // bn, f_out // bf, n_exp // 4)
    return pl.pallas_call(
        _moe_body,
        grid=grid,
        in_specs=[
            pl.BlockSpec((bn, k), lambda i, f, e: (i, 0)),
            pl.BlockSpec((k, n_exp), lambda i, f, e: (0, 0)),
            pl.BlockSpec((1, n_exp), lambda i, f, e: (0, 0)),
            pl.BlockSpec((4, k, bf), lambda i, f, e: (e, 0, f)),
            pl.BlockSpec((n_exp, bf), lambda i, f, e: (0, f)),
        ],
        out_specs=pl.BlockSpec((bn, bf), lambda i, f, e: (i, f)),
        out_shape=jax.ShapeDtypeStruct((n, f_out), jnp.float32),
        scratch_shapes=[
            pltpu.VMEM((bn, n_exp), jnp.float32),
        ],
        compiler_params=pltpu.CompilerParams(
            dimension_semantics=("parallel", "parallel", "arbitrary"),
        ),
    )(x.astype(jnp.bfloat16), Wg.astype(jnp.bfloat16), bg.reshape(1, n_exp),
      We, be)


# speedup vs baseline: 2.3764x; 1.0246x over previous
import jax
import jax.numpy as jnp
from jax.experimental import pallas as pl
from jax.experimental.pallas import tpu as pltpu

_BN = 1024
_BF = 256


def _moe_body(x_ref, wg_ref, bg_ref, we_ref, be_ref, out_ref, g_scr):
    f = pl.program_id(1)
    n_exp = g_scr.shape[1]

    @pl.when(f == 0)
    def _prep():
        logits = jnp.dot(x_ref[...], wg_ref[...],
                         preferred_element_type=jnp.float32)
        logits = logits + bg_ref[...]
        m = jnp.max(logits, axis=-1, keepdims=True)
        p = jnp.exp(logits - m)
        g_scr[...] = p / jnp.sum(p, axis=-1, keepdims=True)

    lane = jax.lax.broadcasted_iota(jnp.int32, (1, n_exp), 1)
    g = g_scr[...]
    xb = x_ref[...]
    contrib = jnp.dot(g, be_ref[...], preferred_element_type=jnp.float32)
    for j in range(8):
        gej = jnp.sum(jnp.where(lane == j, g, 0.0), axis=-1, keepdims=True)
        mmj = jnp.dot(xb, we_ref[j].astype(jnp.bfloat16),
                      preferred_element_type=jnp.float32)
        contrib = contrib + gej * mmj
    out_ref[...] = contrib


def kernel(x, Wg, bg, We, be):
    n, k = x.shape
    n_exp = Wg.shape[1]
    f_out = We.shape[2]
    bn = min(_BN, n)
    bf = min(_BF, f_out)
    grid = (n // bn, f_out // bf)
    return pl.pallas_call(
        _moe_body,
        grid=grid,
        in_specs=[
            pl.BlockSpec((bn, k), lambda i, f: (i, 0)),
            pl.BlockSpec((k, n_exp), lambda i, f: (0, 0)),
            pl.BlockSpec((1, n_exp), lambda i, f: (0, 0)),
            pl.BlockSpec((n_exp, k, bf), lambda i, f: (0, 0, f)),
            pl.BlockSpec((n_exp, bf), lambda i, f: (0, f)),
        ],
        out_specs=pl.BlockSpec((bn, bf), lambda i, f: (i, f)),
        out_shape=jax.ShapeDtypeStruct((n, f_out), jnp.float32),
        scratch_shapes=[
            pltpu.VMEM((bn, n_exp), jnp.float32),
        ],
        compiler_params=pltpu.CompilerParams(
            dimension_semantics=("parallel", "parallel"),
        ),
    )(x.astype(jnp.bfloat16), Wg.astype(jnp.bfloat16), bg.reshape(1, n_exp),
      We, be)


# speedup vs baseline: 2.4518x; 1.0317x over previous
import jax
import jax.numpy as jnp
from jax.experimental import pallas as pl
from jax.experimental.pallas import tpu as pltpu

_BN = 1024
_BF = 256


def _gate_body(x_ref, wg_ref, bg_ref, g_ref, xb_ref):
    xf = x_ref[...]
    logits = jnp.dot(xf, wg_ref[...], preferred_element_type=jnp.float32)
    logits = logits + bg_ref[...]
    m = jnp.max(logits, axis=-1, keepdims=True)
    p = jnp.exp(logits - m)
    g_ref[...] = p / jnp.sum(p, axis=-1, keepdims=True)
    xb_ref[...] = xf.astype(jnp.bfloat16)


def _moe_body(xb_ref, g_ref, we_ref, be_ref, out_ref):
    n_exp = g_ref.shape[1]
    lane = jax.lax.broadcasted_iota(jnp.int32, (1, n_exp), 1)
    g = g_ref[...]
    xb = xb_ref[...]
    contrib = jnp.dot(g, be_ref[...], preferred_element_type=jnp.float32)
    for j in range(n_exp):
        gej = jnp.sum(jnp.where(lane == j, g, 0.0), axis=-1, keepdims=True)
        mmj = jnp.dot(xb, we_ref[j].astype(jnp.bfloat16),
                      preferred_element_type=jnp.float32)
        contrib = contrib + gej * mmj
    out_ref[...] = contrib


def kernel(x, Wg, bg, We, be):
    n, k = x.shape
    n_exp = Wg.shape[1]
    f_out = We.shape[2]
    bn = min(_BN, n)
    bf = min(_BF, f_out)

    g, xb = pl.pallas_call(
        _gate_body,
        grid=(n // bn,),
        in_specs=[
            pl.BlockSpec((bn, k), lambda i: (i, 0)),
            pl.BlockSpec((k, n_exp), lambda i: (0, 0)),
            pl.BlockSpec((1, n_exp), lambda i: (0, 0)),
        ],
        out_specs=[
            pl.BlockSpec((bn, n_exp), lambda i: (i, 0)),
            pl.BlockSpec((bn, k), lambda i: (i, 0)),
        ],
        out_shape=[
            jax.ShapeDtypeStruct((n, n_exp), jnp.float32),
            jax.ShapeDtypeStruct((n, k), jnp.bfloat16),
        ],
        compiler_params=pltpu.CompilerParams(
            dimension_semantics=("parallel",),
        ),
    )(x, Wg, bg.reshape(1, n_exp))

    return pl.pallas_call(
        _moe_body,
        grid=(f_out // bf, n // bn),
        in_specs=[
            pl.BlockSpec((bn, k), lambda f, i: (i, 0)),
            pl.BlockSpec((bn, n_exp), lambda f, i: (i, 0)),
            pl.BlockSpec((n_exp, k, bf), lambda f, i: (0, 0, f)),
            pl.BlockSpec((n_exp, bf), lambda f, i: (0, f)),
        ],
        out_specs=pl.BlockSpec((bn, bf), lambda f, i: (i, f)),
        out_shape=jax.ShapeDtypeStruct((n, f_out), jnp.float32),
        compiler_params=pltpu.CompilerParams(
            dimension_semantics=("parallel", "parallel"),
        ),
    )(xb, g, We, be)
